# baseline (speedup 1.0000x reference)
import functools

import jax
import jax.numpy as jnp
from jax import lax
from jax.experimental import pallas as pl
from jax.experimental.pallas import tpu as pltpu
from jax.experimental.pallas import tpu_sc as plsc

VOCAB = 1000000
EMBED_DIM = 64
PADDED_DIM = 128
BATCH = 4096
SEQ = 200

_info = plsc.get_sparse_core_info()
_NC, _NS, _L = _info.num_cores, _info.num_subcores, _info.num_lanes
_NW = _NC * _NS
_BBLK = BATCH // _NW
_NBUF = 3

_TBLK = 4096


def _tp_body(tt_ref, out_ref):
    out_ref[:, 0:EMBED_DIM] = tt_ref[...].T


@jax.jit
def _transpose_pad(tt):
    grid = (VOCAB + _TBLK - 1) // _TBLK
    return pl.pallas_call(
        _tp_body,
        grid=(grid,),
        in_specs=[pl.BlockSpec((EMBED_DIM, _TBLK), lambda i: (0, i))],
        out_specs=pl.BlockSpec((_TBLK, PADDED_DIM), lambda i: (i, 0)),
        out_shape=jax.ShapeDtypeStruct((VOCAB, PADDED_DIM), jnp.float32),
    )(tt)


def _sc_body(xt_hbm, pos_hbm, tok_hbm, out_hbm, idx_v, pos_v, rows, obufs,
             tmp_v, isem, gsems, ssems):
    wid = lax.axis_index("s") * _NC + lax.axis_index("c")
    col0 = wid * _BBLK

    idx_cp = pltpu.async_copy(
        xt_hbm.at[:, pl.ds(col0, _BBLK)], idx_v, isem)
    pltpu.sync_copy(pos_hbm, pos_v)
    idx_cp.wait()

    def start_gather(s, b):
        pltpu.async_copy(tok_hbm.at[idx_v.at[s]], rows[b], gsems[b])

    def wait_gather(s, b):
        pltpu.make_async_copy(
            tok_hbm.at[idx_v.at[s]], rows[b], gsems[b]).wait()

    def start_store(s, b):
        pltpu.async_copy(
            obufs[b], out_hbm.at[s, :, pl.ds(col0, _BBLK)], ssems[b])

    def wait_store(s, b):
        pltpu.make_async_copy(
            obufs[b], out_hbm.at[s, :, pl.ds(col0, _BBLK)], ssems[b]).wait()

    riota = jax.lax.iota(jnp.int32, _L)
    idx_sk = [(a * _L) + ((riota + a) % _L) for a in range(_L)]
    idx_gt = [(riota * _L) + ((riota + a) % _L) for a in range(_L)]

    def process(s, b):
        rbuf = rows[b]
        obuf = obufs[b]
        pvecs = [pos_v[s, pl.ds(j * _L, _L)] for j in range(EMBED_DIM // _L)]

        def blk(i, carry):
            c0 = i * _L
            for j in range(EMBED_DIM // _L):
                for k in range(_L):
                    v = rbuf[c0 + k, pl.ds(j * _L, _L)] + pvecs[j]
                    plsc.store_scatter(tmp_v, [idx_sk[k]], v)
                for m in range(_L):
                    g = plsc.load_gather(tmp_v, [idx_gt[m]])
                    obuf[j * _L + m, pl.ds(c0, _L)] = g
            return carry

        lax.fori_loop(0, _BBLK // _L, blk, 0)

    start_gather(0, 0)

    def outer(o, carry):
        for b in range(_NBUF):
            s = o * _NBUF + b
            nxt = s + 1
            bn = (b + 1) % _NBUF

            @pl.when(nxt < SEQ)
            def _():
                start_gather(nxt, bn)

            wait_gather(s, b)

            @pl.when(s >= _NBUF)
            def _():
                wait_store(s - _NBUF, b)

            process(s, b)
            start_store(s, b)
        return carry

    lax.fori_loop(0, SEQ // _NBUF, outer, 0)

    for s in range((SEQ // _NBUF) * _NBUF, SEQ):
        b = s % _NBUF
        nxt = s + 1
        bn = (b + 1) % _NBUF
        if nxt < SEQ:
            start_gather(nxt, bn)
        wait_gather(s, b)
        wait_store(s - _NBUF, b)
        process(s, b)
        start_store(s, b)

    for s in range(SEQ - _NBUF, SEQ):
        wait_store(s, s % _NBUF)


@jax.jit
def _embed_sc(xt, token_pad, pos_table):
    mesh = plsc.VectorSubcoreMesh(core_axis_name="c", subcore_axis_name="s")
    k = pl.kernel(
        _sc_body,
        out_type=jax.ShapeDtypeStruct((SEQ, EMBED_DIM, BATCH), jnp.float32),
        mesh=mesh,
        scratch_types=[
            pltpu.VMEM((SEQ, _BBLK), jnp.int32),
            pltpu.VMEM((SEQ, EMBED_DIM), jnp.float32),
            [pltpu.VMEM((_BBLK, PADDED_DIM), jnp.float32)] * _NBUF,
            [pltpu.VMEM((EMBED_DIM, _BBLK), jnp.float32)] * _NBUF,
            pltpu.VMEM((_L * _L,), jnp.float32),
            pltpu.SemaphoreType.DMA,
            [pltpu.SemaphoreType.DMA] * _NBUF,
            [pltpu.SemaphoreType.DMA] * _NBUF,
        ],
        compiler_params=pltpu.CompilerParams(
            use_tc_tiling_on_sc=True, needs_layout_passes=False),
    )
    return k(xt, pos_table, token_pad)


def kernel(x, token_table, pos_table):
    xt = x.astype(jnp.int32).T
    token_pad = _transpose_pad(token_table.T)
    out_t = _embed_sc(xt, token_pad, pos_table)
    return jnp.transpose(out_t, (2, 0, 1))


# speedup vs baseline: 1.4037x; 1.4037x over previous
import functools

import jax
import jax.numpy as jnp
from jax import lax
from jax.experimental import pallas as pl
from jax.experimental.pallas import tpu as pltpu
from jax.experimental.pallas import tpu_sc as plsc

VOCAB = 1000000
EMBED_DIM = 64
PADDED_DIM = 128
BATCH = 4096
SEQ = 200

_info = plsc.get_sparse_core_info()
_NC, _NS, _L = _info.num_cores, _info.num_subcores, _info.num_lanes
_NW = _NC * _NS
_ROWS_PER_W = BATCH // _NW
_NBUF = 3

_TBLK = 4096


def _tp_body(tt_ref, out_ref):
    out_ref[:, 0:EMBED_DIM] = tt_ref[...].T


@jax.jit
def _transpose_pad(tt):
    grid = (VOCAB + _TBLK - 1) // _TBLK
    return pl.pallas_call(
        _tp_body,
        grid=(grid,),
        in_specs=[pl.BlockSpec((EMBED_DIM, _TBLK), lambda i: (0, i))],
        out_specs=pl.BlockSpec((_TBLK, PADDED_DIM), lambda i: (i, 0)),
        out_shape=jax.ShapeDtypeStruct((VOCAB, PADDED_DIM), jnp.float32),
    )(tt)


def _sc_body(x_hbm, pos_hbm, tok_hbm, out_hbm, idx_all, pos_v, rows, isem,
             gsems, ssems):
    wid = lax.axis_index("s") * _NC + lax.axis_index("c")
    row0 = wid * _ROWS_PER_W

    idx_cp = pltpu.async_copy(
        x_hbm.at[pl.ds(row0 * SEQ, _ROWS_PER_W * SEQ)], idx_all, isem)
    pltpu.sync_copy(pos_hbm, pos_v)
    idx_cp.wait()

    def start_gather(chunk, b):
        pltpu.async_copy(
            tok_hbm.at[idx_all.at[pl.ds(chunk * SEQ, SEQ)]], rows[b],
            gsems[b])

    def wait_gather(chunk, b):
        pltpu.make_async_copy(
            tok_hbm.at[idx_all.at[pl.ds(chunk * SEQ, SEQ)]], rows[b],
            gsems[b]).wait()

    def start_store(chunk, b):
        pltpu.async_copy(rows[b], out_hbm.at[row0 + chunk], ssems[b])

    def wait_store(chunk, b):
        pltpu.make_async_copy(
            rows[b], out_hbm.at[row0 + chunk], ssems[b]).wait()

    start_gather(0, 0)

    def outer(o, carry):
        for b in range(_NBUF):
            i = o * _NBUF + b
            nxt = i + 1
            bn = (b + 1) % _NBUF

            @pl.when(nxt < _ROWS_PER_W)
            def _():
                @pl.when(i >= 2)
                def _():
                    wait_store(i - 2, bn)
                start_gather(nxt, bn)

            wait_gather(i, b)

            rbuf = rows[b]

            @plsc.parallel_loop(0, SEQ, unroll=4)
            def _(r):
                for j in range(EMBED_DIM // _L):
                    sl = pl.ds(j * _L, _L)
                    plsc.addupdate(rbuf.at[r, sl], pos_v[r, sl])

            start_store(i, b)
        return carry

    lax.fori_loop(0, _ROWS_PER_W // _NBUF, outer, 0)

    for i in range((_ROWS_PER_W // _NBUF) * _NBUF, _ROWS_PER_W):
        b = i % _NBUF
        nxt = i + 1
        bn = (b + 1) % _NBUF
        if nxt < _ROWS_PER_W:
            wait_store(nxt - 3, bn)
            start_gather(nxt, bn)
        wait_gather(i, b)
        rbuf = rows[b]

        @plsc.parallel_loop(0, SEQ, unroll=4)
        def _(r):
            for j in range(EMBED_DIM // _L):
                sl = pl.ds(j * _L, _L)
                plsc.addupdate(rbuf.at[r, sl], pos_v[r, sl])

        start_store(i, b)

    for i in range(_ROWS_PER_W - _NBUF, _ROWS_PER_W):
        wait_store(i, i % _NBUF)


@jax.jit
def _embed_sc(x_flat, token_pad, pos_table):
    mesh = plsc.VectorSubcoreMesh(core_axis_name="c", subcore_axis_name="s")
    k = pl.kernel(
        _sc_body,
        out_type=jax.ShapeDtypeStruct((BATCH, SEQ, PADDED_DIM), jnp.float32),
        mesh=mesh,
        scratch_types=[
            pltpu.VMEM((_ROWS_PER_W * SEQ,), jnp.int32),
            pltpu.VMEM((SEQ, EMBED_DIM), jnp.float32),
            [pltpu.VMEM((SEQ, PADDED_DIM), jnp.float32)] * _NBUF,
            pltpu.SemaphoreType.DMA,
            [pltpu.SemaphoreType.DMA] * _NBUF,
            [pltpu.SemaphoreType.DMA] * _NBUF,
        ],
        compiler_params=pltpu.CompilerParams(use_tc_tiling_on_sc=True),
    )
    return k(x_flat, pos_table, token_pad)


def kernel(x, token_table, pos_table):
    x_flat = x.reshape(-1).astype(jnp.int32)
    token_pad = _transpose_pad(token_table.T)
    out = _embed_sc(x_flat, token_pad, pos_table)
    return out[:, :, :EMBED_DIM]


# speedup vs baseline: 1.5388x; 1.0963x over previous
import functools

import jax
import jax.numpy as jnp
from jax import lax
from jax.experimental import pallas as pl
from jax.experimental.pallas import tpu as pltpu
from jax.experimental.pallas import tpu_sc as plsc

VOCAB = 1000000
EMBED_DIM = 64
PADDED_DIM = 128
BATCH = 4096
SEQ = 200

_info = plsc.get_sparse_core_info()
_NC, _NS, _L = _info.num_cores, _info.num_subcores, _info.num_lanes
_NW = _NC * _NS
_ROWS_PER_W = BATCH // _NW
_NBUF = 3

_TBLK = 8192


def _tp_body(tt_ref, out_ref):
    out_ref[:, 0:EMBED_DIM] = tt_ref[...].T


@jax.jit
def _transpose_pad(tt):
    grid = (VOCAB + _TBLK - 1) // _TBLK
    return pl.pallas_call(
        _tp_body,
        grid=(grid,),
        in_specs=[pl.BlockSpec((EMBED_DIM, _TBLK), lambda i: (0, i))],
        out_specs=pl.BlockSpec((_TBLK, PADDED_DIM), lambda i: (i, 0)),
        out_shape=jax.ShapeDtypeStruct((VOCAB, PADDED_DIM), jnp.float32),
    )(tt)


def _sc_body(x_hbm, pos_hbm, tok_hbm, out_hbm, idx_all, pos_v, rows, isem,
             gsems, ssems):
    wid = lax.axis_index("s") * _NC + lax.axis_index("c")
    row0 = wid * _ROWS_PER_W

    idx_cp = pltpu.async_copy(
        x_hbm.at[pl.ds(row0 * SEQ, _ROWS_PER_W * SEQ)], idx_all, isem)
    pltpu.sync_copy(pos_hbm, pos_v)
    idx_cp.wait()

    def start_gather(chunk, b):
        pltpu.async_copy(
            tok_hbm.at[idx_all.at[pl.ds(chunk * SEQ, SEQ)]], rows[b],
            gsems[b])

    def wait_gather(chunk, b):
        pltpu.make_async_copy(
            tok_hbm.at[idx_all.at[pl.ds(chunk * SEQ, SEQ)]], rows[b],
            gsems[b]).wait()

    def start_store(chunk, b):
        pltpu.async_copy(rows[b], out_hbm.at[row0 + chunk], ssems[b])

    def wait_store(chunk, b):
        pltpu.make_async_copy(
            rows[b], out_hbm.at[row0 + chunk], ssems[b]).wait()

    start_gather(0, 0)

    def outer(o, carry):
        for b in range(_NBUF):
            i = o * _NBUF + b
            nxt = i + 1
            bn = (b + 1) % _NBUF

            @pl.when(nxt < _ROWS_PER_W)
            def _():
                @pl.when(i >= 2)
                def _():
                    wait_store(i - 2, bn)
                start_gather(nxt, bn)

            wait_gather(i, b)

            rbuf = rows[b]

            @plsc.parallel_loop(0, SEQ, unroll=4)
            def _(r):
                for j in range(EMBED_DIM // _L):
                    sl = pl.ds(j * _L, _L)
                    plsc.addupdate(rbuf.at[r, sl], pos_v[r, sl])

            start_store(i, b)
        return carry

    lax.fori_loop(0, _ROWS_PER_W // _NBUF, outer, 0)

    for i in range((_ROWS_PER_W // _NBUF) * _NBUF, _ROWS_PER_W):
        b = i % _NBUF
        nxt = i + 1
        bn = (b + 1) % _NBUF
        if nxt < _ROWS_PER_W:
            wait_store(nxt - 3, bn)
            start_gather(nxt, bn)
        wait_gather(i, b)
        rbuf = rows[b]

        @plsc.parallel_loop(0, SEQ, unroll=4)
        def _(r):
            for j in range(EMBED_DIM // _L):
                sl = pl.ds(j * _L, _L)
                plsc.addupdate(rbuf.at[r, sl], pos_v[r, sl])

        start_store(i, b)

    for i in range(_ROWS_PER_W - _NBUF, _ROWS_PER_W):
        wait_store(i, i % _NBUF)


@jax.jit
def _embed_sc(x_flat, token_pad, pos_table):
    mesh = plsc.VectorSubcoreMesh(core_axis_name="c", subcore_axis_name="s")
    k = pl.kernel(
        _sc_body,
        out_type=jax.ShapeDtypeStruct((BATCH, SEQ, PADDED_DIM), jnp.float32),
        mesh=mesh,
        scratch_types=[
            pltpu.VMEM((_ROWS_PER_W * SEQ,), jnp.int32),
            pltpu.VMEM((SEQ, EMBED_DIM), jnp.float32),
            [pltpu.VMEM((SEQ, PADDED_DIM), jnp.float32)] * _NBUF,
            pltpu.SemaphoreType.DMA,
            [pltpu.SemaphoreType.DMA] * _NBUF,
            [pltpu.SemaphoreType.DMA] * _NBUF,
        ],
        compiler_params=pltpu.CompilerParams(use_tc_tiling_on_sc=True),
    )
    return k(x_flat, pos_table, token_pad)


def kernel(x, token_table, pos_table):
    x_flat = x.reshape(-1).astype(jnp.int32)
    token_pad = _transpose_pad(token_table.T)
    out = _embed_sc(x_flat, token_pad, pos_table)
    return out[:, :, :EMBED_DIM]


# speedup vs baseline: 1.5819x; 1.0280x over previous
import functools

import jax
import jax.numpy as jnp
from jax import lax
from jax.experimental import pallas as pl
from jax.experimental.pallas import tpu as pltpu
from jax.experimental.pallas import tpu_sc as plsc

VOCAB = 1000000
EMBED_DIM = 64
PADDED_DIM = 128
BATCH = 4096
SEQ = 200

_info = plsc.get_sparse_core_info()
_NC, _NS, _L = _info.num_cores, _info.num_subcores, _info.num_lanes
_NW = _NC * _NS
_ROWS_PER_W = BATCH // _NW
_NBUF = 3

_TBLK = 16384


def _tp_body(tt_ref, out_ref):
    out_ref[:, 0:EMBED_DIM] = tt_ref[...].T


@jax.jit
def _transpose_pad(tt):
    grid = (VOCAB + _TBLK - 1) // _TBLK
    return pl.pallas_call(
        _tp_body,
        grid=(grid,),
        in_specs=[pl.BlockSpec((EMBED_DIM, _TBLK), lambda i: (0, i))],
        out_specs=pl.BlockSpec((_TBLK, PADDED_DIM), lambda i: (i, 0)),
        out_shape=jax.ShapeDtypeStruct((VOCAB, PADDED_DIM), jnp.float32),
    )(tt)


def _sc_body(x_hbm, pos_hbm, tok_hbm, out_hbm, idx_all, pos_v, rows, isem,
             gsems, ssems):
    wid = lax.axis_index("s") * _NC + lax.axis_index("c")
    row0 = wid * _ROWS_PER_W

    idx_cp = pltpu.async_copy(
        x_hbm.at[pl.ds(row0 * SEQ, _ROWS_PER_W * SEQ)], idx_all, isem)
    pltpu.sync_copy(pos_hbm, pos_v)
    idx_cp.wait()

    def start_gather(chunk, b):
        pltpu.async_copy(
            tok_hbm.at[idx_all.at[pl.ds(chunk * SEQ, SEQ)]], rows[b],
            gsems[b])

    def wait_gather(chunk, b):
        pltpu.make_async_copy(
            tok_hbm.at[idx_all.at[pl.ds(chunk * SEQ, SEQ)]], rows[b],
            gsems[b]).wait()

    def start_store(chunk, b):
        pltpu.async_copy(rows[b], out_hbm.at[row0 + chunk], ssems[b])

    def wait_store(chunk, b):
        pltpu.make_async_copy(
            rows[b], out_hbm.at[row0 + chunk], ssems[b]).wait()

    start_gather(0, 0)

    def outer(o, carry):
        for b in range(_NBUF):
            i = o * _NBUF + b
            nxt = i + 1
            bn = (b + 1) % _NBUF

            @pl.when(nxt < _ROWS_PER_W)
            def _():
                @pl.when(i >= 2)
                def _():
                    wait_store(i - 2, bn)
                start_gather(nxt, bn)

            wait_gather(i, b)

            rbuf = rows[b]

            @plsc.parallel_loop(0, SEQ, unroll=4)
            def _(r):
                for j in range(EMBED_DIM // _L):
                    sl = pl.ds(j * _L, _L)
                    plsc.addupdate(rbuf.at[r, sl], pos_v[r, sl])

            start_store(i, b)
        return carry

    lax.fori_loop(0, _ROWS_PER_W // _NBUF, outer, 0)

    for i in range((_ROWS_PER_W // _NBUF) * _NBUF, _ROWS_PER_W):
        b = i % _NBUF
        nxt = i + 1
        bn = (b + 1) % _NBUF
        if nxt < _ROWS_PER_W:
            wait_store(nxt - 3, bn)
            start_gather(nxt, bn)
        wait_gather(i, b)
        rbuf = rows[b]

        @plsc.parallel_loop(0, SEQ, unroll=4)
        def _(r):
            for j in range(EMBED_DIM // _L):
                sl = pl.ds(j * _L, _L)
                plsc.addupdate(rbuf.at[r, sl], pos_v[r, sl])

        start_store(i, b)

    for i in range(_ROWS_PER_W - _NBUF, _ROWS_PER_W):
        wait_store(i, i % _NBUF)


@jax.jit
def _embed_sc(x_flat, token_pad, pos_table):
    mesh = plsc.VectorSubcoreMesh(core_axis_name="c", subcore_axis_name="s")
    k = pl.kernel(
        _sc_body,
        out_type=jax.ShapeDtypeStruct((BATCH, SEQ, PADDED_DIM), jnp.float32),
        mesh=mesh,
        scratch_types=[
            pltpu.VMEM((_ROWS_PER_W * SEQ,), jnp.int32),
            pltpu.VMEM((SEQ, EMBED_DIM), jnp.float32),
            [pltpu.VMEM((SEQ, PADDED_DIM), jnp.float32)] * _NBUF,
            pltpu.SemaphoreType.DMA,
            [pltpu.SemaphoreType.DMA] * _NBUF,
            [pltpu.SemaphoreType.DMA] * _NBUF,
        ],
        compiler_params=pltpu.CompilerParams(use_tc_tiling_on_sc=True),
    )
    return k(x_flat, pos_table, token_pad)


def kernel(x, token_table, pos_table):
    x_flat = x.reshape(-1).astype(jnp.int32)
    token_pad = _transpose_pad(token_table.T)
    out = _embed_sc(x_flat, token_pad, pos_table)
    return out[:, :, :EMBED_DIM]


# speedup vs baseline: 1.5918x; 1.0063x over previous
import functools

import jax
import jax.numpy as jnp
from jax import lax
from jax.experimental import pallas as pl
from jax.experimental.pallas import tpu as pltpu
from jax.experimental.pallas import tpu_sc as plsc

VOCAB = 1000000
EMBED_DIM = 64
PADDED_DIM = 128
BATCH = 4096
SEQ = 200

_info = plsc.get_sparse_core_info()
_NC, _NS, _L = _info.num_cores, _info.num_subcores, _info.num_lanes
_NW = _NC * _NS
_ROWS_PER_W = BATCH // _NW
_NBUF = 3

_TBLK = 32768


def _tp_body(tt_ref, out_ref):
    out_ref[:, 0:EMBED_DIM] = tt_ref[...].T


@jax.jit
def _transpose_pad(tt):
    grid = (VOCAB + _TBLK - 1) // _TBLK
    return pl.pallas_call(
        _tp_body,
        grid=(grid,),
        in_specs=[pl.BlockSpec((EMBED_DIM, _TBLK), lambda i: (0, i))],
        out_specs=pl.BlockSpec((_TBLK, PADDED_DIM), lambda i: (i, 0)),
        out_shape=jax.ShapeDtypeStruct((VOCAB, PADDED_DIM), jnp.float32),
    )(tt)


def _sc_body(x_hbm, pos_hbm, tok_hbm, out_hbm, idx_all, pos_v, rows, isem,
             gsems, ssems):
    wid = lax.axis_index("s") * _NC + lax.axis_index("c")
    row0 = wid * _ROWS_PER_W

    idx_cp = pltpu.async_copy(
        x_hbm.at[pl.ds(row0 * SEQ, _ROWS_PER_W * SEQ)], idx_all, isem)
    pltpu.sync_copy(pos_hbm, pos_v)
    idx_cp.wait()

    def start_gather(chunk, b):
        pltpu.async_copy(
            tok_hbm.at[idx_all.at[pl.ds(chunk * SEQ, SEQ)]], rows[b],
            gsems[b])

    def wait_gather(chunk, b):
        pltpu.make_async_copy(
            tok_hbm.at[idx_all.at[pl.ds(chunk * SEQ, SEQ)]], rows[b],
            gsems[b]).wait()

    def start_store(chunk, b):
        pltpu.async_copy(rows[b], out_hbm.at[row0 + chunk], ssems[b])

    def wait_store(chunk, b):
        pltpu.make_async_copy(
            rows[b], out_hbm.at[row0 + chunk], ssems[b]).wait()

    start_gather(0, 0)

    def outer(o, carry):
        for b in range(_NBUF):
            i = o * _NBUF + b
            nxt = i + 1
            bn = (b + 1) % _NBUF

            @pl.when(nxt < _ROWS_PER_W)
            def _():
                @pl.when(i >= 2)
                def _():
                    wait_store(i - 2, bn)
                start_gather(nxt, bn)

            wait_gather(i, b)

            rbuf = rows[b]

            @plsc.parallel_loop(0, SEQ, unroll=4)
            def _(r):
                for j in range(EMBED_DIM // _L):
                    sl = pl.ds(j * _L, _L)
                    plsc.addupdate(rbuf.at[r, sl], pos_v[r, sl])

            start_store(i, b)
        return carry

    lax.fori_loop(0, _ROWS_PER_W // _NBUF, outer, 0)

    for i in range((_ROWS_PER_W // _NBUF) * _NBUF, _ROWS_PER_W):
        b = i % _NBUF
        nxt = i + 1
        bn = (b + 1) % _NBUF
        if nxt < _ROWS_PER_W:
            wait_store(nxt - 3, bn)
            start_gather(nxt, bn)
        wait_gather(i, b)
        rbuf = rows[b]

        @plsc.parallel_loop(0, SEQ, unroll=4)
        def _(r):
            for j in range(EMBED_DIM // _L):
                sl = pl.ds(j * _L, _L)
                plsc.addupdate(rbuf.at[r, sl], pos_v[r, sl])

        start_store(i, b)

    for i in range(_ROWS_PER_W - _NBUF, _ROWS_PER_W):
        wait_store(i, i % _NBUF)


@jax.jit
def _embed_sc(x_flat, token_pad, pos_table):
    mesh = plsc.VectorSubcoreMesh(core_axis_name="c", subcore_axis_name="s")
    k = pl.kernel(
        _sc_body,
        out_type=jax.ShapeDtypeStruct((BATCH, SEQ, PADDED_DIM), jnp.float32),
        mesh=mesh,
        scratch_types=[
            pltpu.VMEM((_ROWS_PER_W * SEQ,), jnp.int32),
            pltpu.VMEM((SEQ, EMBED_DIM), jnp.float32),
            [pltpu.VMEM((SEQ, PADDED_DIM), jnp.float32)] * _NBUF,
            pltpu.SemaphoreType.DMA,
            [pltpu.SemaphoreType.DMA] * _NBUF,
            [pltpu.SemaphoreType.DMA] * _NBUF,
        ],
        compiler_params=pltpu.CompilerParams(use_tc_tiling_on_sc=True),
    )
    return k(x_flat, pos_table, token_pad)


def kernel(x, token_table, pos_table):
    x_flat = x.reshape(-1).astype(jnp.int32)
    token_pad = _transpose_pad(token_table.T)
    out = _embed_sc(x_flat, token_pad, pos_table)
    return out[:, :, :EMBED_DIM]
